# baseline (speedup 1.0000x reference)
import functools

import jax
import jax.numpy as jnp
from jax import lax
from jax.experimental import pallas as pl
from jax.experimental.pallas import tpu as pltpu
from jax.experimental.pallas import tpu_sc as plsc

NUM_CORES = 2
NUM_SUBCORES = 16
NUM_WORKERS = NUM_CORES * NUM_SUBCORES
BATCH = 16384
DIM = 64
B_PER_W = BATCH // NUM_WORKERS
LANES = 16
EG = 4
SUBS = LANES // EG
BLOCKS = B_PER_W // LANES


@functools.partial(
    pl.kernel,
    mesh=plsc.VectorSubcoreMesh(core_axis_name="c", subcore_axis_name="s"),
    out_type=jax.ShapeDtypeStruct((BATCH,), jnp.float32),
    compiler_params=pltpu.CompilerParams(needs_layout_passes=False),
    scratch_types=[
        pltpu.VMEM((B_PER_W,), jnp.int32),
        pltpu.VMEM((B_PER_W,), jnp.int32),
        pltpu.VMEM((EG, DIM, 128), jnp.float32),
        pltpu.VMEM((EG, DIM, 128), jnp.float32),
        pltpu.VMEM((B_PER_W,), jnp.float32),
        pltpu.SemaphoreType.DMA((EG,)),
        pltpu.SemaphoreType.DMA((EG,)),
    ],
)
def _wmf_sc(uidx_hbm, iidx_hbm, utab_hbm, itab_hbm, out_hbm,
            uidx_v, iidx_v, ublk_v, iblk_v, out_v, usem, isem):
    wid = lax.axis_index("s") * NUM_CORES + lax.axis_index("c")
    base = wid * B_PER_W

    pltpu.sync_copy(uidx_hbm.at[wid], uidx_v)
    pltpu.sync_copy(iidx_hbm.at[wid], iidx_v)

    iota16 = lax.iota(jnp.int32, 16)

    def block_body(b, carry):
        uvec = uidx_v[pl.ds(b * LANES, LANES)]
        ivec = iidx_v[pl.ds(b * LANES, LANES)]
        ublks = (uvec // 128) * 128
        iblks = (ivec // 128) * 128
        ucols = uvec - ublks
        icols = ivec - iblks
        acc = jnp.zeros((LANES,), jnp.float32)
        for s in range(SUBS):
            copies = []
            for e in range(EG):
                ub = pl.multiple_of(ublks[s * EG + e], 128)
                ib = pl.multiple_of(iblks[s * EG + e], 128)
                copies.append(pltpu.async_copy(
                    utab_hbm.at[:, pl.ds(ub, 128)], ublk_v.at[e], usem.at[e]))
                copies.append(pltpu.async_copy(
                    itab_hbm.at[:, pl.ds(ib, 128)], iblk_v.at[e], isem.at[e]))
            for e in range(EG):
                copies[2 * e].wait()
                copies[2 * e + 1].wait()
                le = s * EG + e
                ucol = jnp.full((LANES,), ucols[le], jnp.int32)
                icol = jnp.full((LANES,), icols[le], jnp.int32)
                prod = jnp.zeros((LANES,), jnp.float32)
                for k in range(DIM // LANES):
                    rows = k * LANES + iota16
                    uu = plsc.load_gather(ublk_v.at[e], [rows, ucol])
                    vv = plsc.load_gather(iblk_v.at[e], [rows, icol])
                    prod = prod + uu * vv
                dot = lax.reduce_sum_p.bind(prod, axes=(0,))
                acc = jnp.where(iota16 == le, dot, acc)
        out_v[pl.ds(b * LANES, LANES)] = 1.0 / (1.0 + jnp.exp(-acc))
        return carry

    lax.fori_loop(0, BLOCKS, block_body, 0)

    pltpu.sync_copy(out_v, out_hbm.at[pl.ds(base, B_PER_W)])


def kernel(user_indices, item_indices, user_table, item_table):
    uidx = user_indices.astype(jnp.int32).reshape(NUM_WORKERS, B_PER_W)
    iidx = item_indices.astype(jnp.int32).reshape(NUM_WORKERS, B_PER_W)
    return _wmf_sc(uidx, iidx, user_table.T, item_table.T)


# speedup vs baseline: 1.0493x; 1.0493x over previous
import functools

import jax
import jax.numpy as jnp
from jax import lax
from jax.experimental import pallas as pl
from jax.experimental.pallas import tpu as pltpu
from jax.experimental.pallas import tpu_sc as plsc

NUM_CORES = 2
NUM_SUBCORES = 16
NUM_WORKERS = NUM_CORES * NUM_SUBCORES
BATCH = 16384
DIM = 64
B_PER_W = BATCH // NUM_WORKERS
LANES = 16
EG = 4
SUBS = LANES // EG
BLOCKS = B_PER_W // LANES


@functools.partial(
    pl.kernel,
    mesh=plsc.VectorSubcoreMesh(core_axis_name="c", subcore_axis_name="s"),
    out_type=jax.ShapeDtypeStruct((BATCH,), jnp.float32),
    compiler_params=pltpu.CompilerParams(needs_layout_passes=False),
    scratch_types=[
        pltpu.VMEM((B_PER_W,), jnp.int32),
        pltpu.VMEM((B_PER_W,), jnp.int32),
        pltpu.VMEM((EG, DIM, 128), jnp.float32),
        pltpu.VMEM((EG, DIM, 128), jnp.float32),
        pltpu.VMEM((B_PER_W,), jnp.float32),
        pltpu.SemaphoreType.DMA,
    ],
)
def _wmf_sc(uidx_hbm, iidx_hbm, utab_hbm, itab_hbm, out_hbm,
            uidx_v, iidx_v, ublk_v, iblk_v, out_v, sem):
    wid = lax.axis_index("s") * NUM_CORES + lax.axis_index("c")
    base = wid * B_PER_W

    pltpu.sync_copy(uidx_hbm.at[wid], uidx_v)
    pltpu.sync_copy(iidx_hbm.at[wid], iidx_v)

    iota16 = lax.iota(jnp.int32, 16)

    def block_body(b, carry):
        uvec = uidx_v[pl.ds(b * LANES, LANES)]
        ivec = iidx_v[pl.ds(b * LANES, LANES)]
        ublks = (uvec // 128) * 128
        iblks = (ivec // 128) * 128
        ucols = uvec - ublks
        icols = ivec - iblks
        acc = jnp.zeros((LANES,), jnp.float32)
        for s in range(SUBS):
            copies = []
            for e in range(EG):
                ub = pl.multiple_of(ublks[s * EG + e], 128)
                ib = pl.multiple_of(iblks[s * EG + e], 128)
                copies.append(pltpu.async_copy(
                    utab_hbm.at[:, pl.ds(ub, 128)], ublk_v.at[e], sem))
                copies.append(pltpu.async_copy(
                    itab_hbm.at[:, pl.ds(ib, 128)], iblk_v.at[e], sem))
            for c in copies:
                c.wait()
            for e in range(EG):
                le = s * EG + e
                ucol = jnp.full((LANES,), ucols[le], jnp.int32)
                icol = jnp.full((LANES,), icols[le], jnp.int32)
                prod = jnp.zeros((LANES,), jnp.float32)
                for k in range(DIM // LANES):
                    rows = k * LANES + iota16
                    uu = plsc.load_gather(ublk_v.at[e], [rows, ucol])
                    vv = plsc.load_gather(iblk_v.at[e], [rows, icol])
                    prod = prod + uu * vv
                dot = lax.reduce_sum_p.bind(prod, axes=(0,))
                acc = jnp.where(iota16 == le, dot, acc)
        out_v[pl.ds(b * LANES, LANES)] = 1.0 / (1.0 + jnp.exp(-acc))
        return carry

    lax.fori_loop(0, BLOCKS, block_body, 0)

    pltpu.sync_copy(out_v, out_hbm.at[pl.ds(base, B_PER_W)])


def kernel(user_indices, item_indices, user_table, item_table):
    uidx = user_indices.astype(jnp.int32).reshape(NUM_WORKERS, B_PER_W)
    iidx = item_indices.astype(jnp.int32).reshape(NUM_WORKERS, B_PER_W)
    return _wmf_sc(uidx, iidx, user_table.T, item_table.T)
